# baseline (speedup 1.0000x reference)
import jax
import jax.numpy as jnp
from jax.experimental import pallas as pl
from jax.experimental.pallas import tpu as pltpu


def _mm_kernel(idx_ref, x_ref, w_hbm, b_ref, o_ref, wscr, wsem):
    E = wscr.shape[0]
    G, S = idx_ref.shape
    P = G * S
    i = pl.program_id(0)

    R = 2
    RE = E // R

    def _flat(p):
        return idx_ref[p // S, p % S]

    def _wcopy(r):
        return pltpu.make_async_copy(
            w_hbm.at[pl.ds(r * RE, RE)], wscr.at[pl.ds(r * RE, RE)],
            wsem.at[r])

    @pl.when(i == 0)
    def _issue():
        first_run = _flat(0) // RE
        _wcopy(first_run).start()
        _wcopy(1 - first_run).start()

    xa = x_ref[0]
    for s in range(S):
        e_s = idx_ref[i, s]
        r_s = e_s // RE
        p = i * S + s
        fu = jnp.bool_(True)
        for q in range(P):
            clash = jnp.logical_and(q < p, _flat(q) // RE == r_s)
            fu = jnp.logical_and(fu, jnp.logical_not(clash))

        @pl.when(fu)
        def _wait_w():
            _wcopy(r_s).wait()

        acc = jax.lax.dot_general(
            xa, wscr[e_s],
            dimension_numbers=(((1,), (1,)), ((), ())),
            preferred_element_type=jnp.float32,
        )
        o_ref[0, s] = acc + b_ref[e_s]

    @pl.when(i == pl.num_programs(0) - 1)
    def _drain_unused():
        for r in range(R):
            used = jnp.bool_(False)
            for q in range(P):
                used = jnp.logical_or(used, _flat(q) // RE == r)

            @pl.when(jnp.logical_not(used))
            def _wait_unused():
                _wcopy(r).wait()


def kernel(x, indices, weight, bias):
    G, S = indices.shape
    E, OUT_D, IN_D = weight.shape
    T = x.shape[-2]

    xr = x.reshape(G, T, IN_D)

    grid_spec = pltpu.PrefetchScalarGridSpec(
        num_scalar_prefetch=1,
        grid=(G,),
        in_specs=[
            pl.BlockSpec((1, T, IN_D), lambda i, ind: (i, 0, 0)),
            pl.BlockSpec(memory_space=pl.ANY),
            pl.BlockSpec((E, OUT_D), lambda i, ind: (0, 0)),
        ],
        out_specs=pl.BlockSpec((1, S, T, OUT_D),
                               lambda i, ind: (i, 0, 0, 0)),
        scratch_shapes=[
            pltpu.VMEM((E, OUT_D, IN_D), jnp.float32),
            pltpu.SemaphoreType.DMA((E,)),
        ],
    )

    out = pl.pallas_call(
        _mm_kernel,
        grid_spec=grid_spec,
        out_shape=jax.ShapeDtypeStruct((G, S, T, OUT_D), jnp.float32),
    )(indices, xr, weight, bias)

    return out.reshape(1, G, S, T, OUT_D)


# speedup vs baseline: 1.1904x; 1.1904x over previous
import jax
import jax.numpy as jnp
from jax.experimental import pallas as pl
from jax.experimental.pallas import tpu as pltpu


def _mm_kernel(idx_ref, x_ref, w_hbm, b_ref, o_ref, wscr, wsem):
    E = wscr.shape[0]
    G, S = idx_ref.shape
    P = G * S
    i = pl.program_id(0)

    def _flat(p):
        return idx_ref[p // S, p % S]

    def _wcopy(e):
        return pltpu.make_async_copy(w_hbm.at[e], wscr.at[e], wsem.at[e])

    @pl.when(i == 0)
    def _issue():
        for p in range(P):
            e = _flat(p)
            fu = jnp.bool_(True)
            for q in range(p):
                fu = jnp.logical_and(fu, _flat(q) != e)

            @pl.when(fu)
            def _start_w():
                _wcopy(e).start()

    xa = x_ref[0]
    for s in range(S):
        e_s = idx_ref[i, s]
        p = i * S + s
        fu = jnp.bool_(True)
        for q in range(P):
            clash = jnp.logical_and(q < p, _flat(q) == e_s)
            fu = jnp.logical_and(fu, jnp.logical_not(clash))

        @pl.when(fu)
        def _wait_w():
            _wcopy(e_s).wait()

        acc = jax.lax.dot_general(
            xa, wscr[e_s],
            dimension_numbers=(((1,), (1,)), ((), ())),
            preferred_element_type=jnp.float32,
        )
        o_ref[0, s] = acc + b_ref[e_s]


def kernel(x, indices, weight, bias):
    G, S = indices.shape
    E, OUT_D, IN_D = weight.shape
    T = x.shape[-2]

    xr = x.reshape(G, T, IN_D)

    grid_spec = pltpu.PrefetchScalarGridSpec(
        num_scalar_prefetch=1,
        grid=(G,),
        in_specs=[
            pl.BlockSpec((1, T, IN_D), lambda i, ind: (i, 0, 0)),
            pl.BlockSpec(memory_space=pl.ANY),
            pl.BlockSpec((E, OUT_D), lambda i, ind: (0, 0)),
        ],
        out_specs=pl.BlockSpec((1, S, T, OUT_D),
                               lambda i, ind: (i, 0, 0, 0)),
        scratch_shapes=[
            pltpu.VMEM((E, OUT_D, IN_D), jnp.float32),
            pltpu.SemaphoreType.DMA((E,)),
        ],
    )

    out = pl.pallas_call(
        _mm_kernel,
        grid_spec=grid_spec,
        out_shape=jax.ShapeDtypeStruct((G, S, T, OUT_D), jnp.float32),
    )(indices, xr, weight, bias)

    return out.reshape(1, G, S, T, OUT_D)


# speedup vs baseline: 1.1911x; 1.0006x over previous
import jax
import jax.numpy as jnp
from jax.experimental import pallas as pl
from jax.experimental.pallas import tpu as pltpu


def _mm_kernel(idx_ref, x_ref, w_hbm, b_ref, o_ref, wscr, wsem):
    E = wscr.shape[0]
    G, S = idx_ref.shape
    P = G * S
    i = pl.program_id(0)

    def _flat(p):
        return idx_ref[p // S, p % S]

    def _wcopy(e):
        return pltpu.make_async_copy(w_hbm.at[e], wscr.at[e], wsem.at[e])

    @pl.when(i == 0)
    def _issue():
        for p in range(P):
            e = _flat(p)
            fu = jnp.bool_(True)
            for q in range(p):
                fu = jnp.logical_and(fu, _flat(q) != e)

            @pl.when(fu)
            def _start_w():
                _wcopy(e).start()

    xa = x_ref[0, 0, 0]
    for s in range(S):
        e_s = idx_ref[i, s]
        p = i * S + s
        fu = jnp.bool_(True)
        for q in range(P):
            clash = jnp.logical_and(q < p, _flat(q) == e_s)
            fu = jnp.logical_and(fu, jnp.logical_not(clash))

        @pl.when(fu)
        def _wait_w():
            _wcopy(e_s).wait()

        acc = jax.lax.dot_general(
            xa, wscr[e_s],
            dimension_numbers=(((1,), (1,)), ((), ())),
            preferred_element_type=jnp.float32,
        )
        o_ref[0, 0, s] = acc + b_ref[e_s]


def kernel(x, indices, weight, bias):
    G, S = indices.shape
    E, OUT_D, IN_D = weight.shape
    T = x.shape[-2]

    grid_spec = pltpu.PrefetchScalarGridSpec(
        num_scalar_prefetch=1,
        grid=(G,),
        in_specs=[
            pl.BlockSpec((1, 1, 1, T, IN_D), lambda i, ind: (0, i, 0, 0, 0)),
            pl.BlockSpec(memory_space=pl.ANY),
            pl.BlockSpec((E, OUT_D), lambda i, ind: (0, 0)),
        ],
        out_specs=pl.BlockSpec((1, 1, S, T, OUT_D),
                               lambda i, ind: (0, i, 0, 0, 0)),
        scratch_shapes=[
            pltpu.VMEM((E, OUT_D, IN_D), jnp.float32),
            pltpu.SemaphoreType.DMA((E,)),
        ],
    )

    return pl.pallas_call(
        _mm_kernel,
        grid_spec=grid_spec,
        out_shape=jax.ShapeDtypeStruct((1, G, S, T, OUT_D), jnp.float32),
    )(indices, x, weight, bias)
